# baseline (speedup 1.0000x reference)
import jax
import jax.numpy as jnp
from jax import lax
from jax.experimental import pallas as pl
from jax.experimental.pallas import tpu as pltpu
from jax.experimental.pallas import tpu_sc as plsc

N = 10000
E = 320000
D = 128
NV = D // 16

NC = 2
NS = 16
NW = NC * NS
RPW = 312
RLAST = N - (NW - 1) * RPW

SCN = 1600
NCHK = E // SCN
SEG = 10
NSEG = NCHK // SEG
WC = 80
KCH = 4
LR = 2064
LCAP = KCH * LR
RB = 3


def _mm_body(x_ref, w_ref, o_ref):
    o_ref[...] = jnp.dot(x_ref[...], w_ref[...],
                         preferred_element_type=jnp.float32)


def _matmul(x, W, bm=2000):
    n, k = x.shape
    m = W.shape[1]
    return pl.pallas_call(
        _mm_body,
        grid=(n // bm,),
        in_specs=[pl.BlockSpec((bm, k), lambda i: (i, 0)),
                  pl.BlockSpec((k, m), lambda i: (0, 0))],
        out_specs=pl.BlockSpec((bm, m), lambda i: (i, 0)),
        out_shape=jax.ShapeDtypeStruct((n, m), jnp.float32),
    )(x, W)


def _fused_body(p_ref, b_ref, w_ref, o_ref):
    h = jnp.maximum(p_ref[...] + b_ref[...], 0.0)
    o_ref[...] = jnp.dot(h, w_ref[...], preferred_element_type=jnp.float32)


def _fused_relu_mm(p, b, W, bm=2000):
    n, k = p.shape
    m = W.shape[1]
    return pl.pallas_call(
        _fused_body,
        grid=(n // bm,),
        in_specs=[pl.BlockSpec((bm, k), lambda i: (i, 0)),
                  pl.BlockSpec((1, k), lambda i: (0, 0)),
                  pl.BlockSpec((k, m), lambda i: (0, 0))],
        out_specs=pl.BlockSpec((bm, m), lambda i: (i, 0)),
        out_shape=jax.ShapeDtypeStruct((n, m), jnp.float32),
    )(p, b.reshape(1, k), W)


def _bias_body(p_ref, b_ref, o_ref):
    o_ref[...] = p_ref[...] + b_ref[...]


def _add_bias(p, b, bm=2000):
    n, k = p.shape
    return pl.pallas_call(
        _bias_body,
        grid=(n // bm,),
        in_specs=[pl.BlockSpec((bm, k), lambda i: (i, 0)),
                  pl.BlockSpec((1, k), lambda i: (0, 0))],
        out_specs=pl.BlockSpec((bm, k), lambda i: (i, 0)),
        out_shape=jax.ShapeDtypeStruct((n, k), jnp.float32),
    )(p, b.reshape(1, k))


def _agg_body(h_hbm, src_hbm, dst_hbm, w_hbm, out_hbm,
              sbuf, dbuf, wbuf, lsrc, ldl, lw, rowbuf, acc, isem, gsem):
    c = lax.axis_index("c")
    s = lax.axis_index("s")
    wid = c * NS + s

    lo = wid * RPW
    rows_own = jnp.where(wid == NW - 1, RLAST, RPW).astype(jnp.int32)
    hi = lo + rows_own

    def _zrow(r, carry):
        zero = jnp.zeros((16,), jnp.float32)
        for j in range(NV):
            acc[r, pl.ds(j * 16, 16)] = zero
        return carry
    lax.fori_loop(0, RLAST, _zrow, None)

    def fire_scan(k):
        sl = lax.rem(k, 2)
        base = pl.multiple_of(k * SCN, 8)
        sb = pl.multiple_of(sl * SCN, 8)
        pltpu.async_copy(src_hbm.at[pl.ds(base, SCN)],
                         sbuf.at[pl.ds(sb, SCN)], isem.at[sl])
        pltpu.async_copy(dst_hbm.at[pl.ds(base, SCN)],
                         dbuf.at[pl.ds(sb, SCN)], isem.at[sl])
        pltpu.async_copy(w_hbm.at[pl.ds(base, SCN)],
                         wbuf.at[pl.ds(sb, SCN)], isem.at[sl])

    def wait_scan(k):
        sl = lax.rem(k, 2)
        sb = pl.multiple_of(sl * SCN, 8)
        pltpu.make_async_copy(src_hbm.at[pl.ds(0, SCN)],
                              sbuf.at[pl.ds(sb, SCN)], isem.at[sl]).wait()
        pltpu.make_async_copy(dst_hbm.at[pl.ds(0, SCN)],
                              dbuf.at[pl.ds(sb, SCN)], isem.at[sl]).wait()
        pltpu.make_async_copy(w_hbm.at[pl.ds(0, SCN)],
                              wbuf.at[pl.ds(sb, SCN)], isem.at[sl]).wait()

    def fire_gather(p, base):
        ri = lax.rem(p, RB)
        pltpu.async_copy(h_hbm.at[lsrc.at[pl.ds(base, WC)]], rowbuf.at[ri],
                         gsem.at[ri])

    def wait_gather(p):
        ri = lax.rem(p, RB)
        pltpu.make_async_copy(h_hbm.at[lsrc.at[pl.ds(0, WC)]], rowbuf.at[ri],
                              gsem.at[ri]).wait()

    def process_window(p, base):
        ri = lax.rem(p, RB)

        def _g16(q, carry):
            lb = base + q * 16
            w16 = lw[pl.ds(lb, 16)]
            l16 = ldl[pl.ds(lb, 16)]
            for i in range(16):
                wv = jnp.full((16,), w16[i], jnp.float32)
                lrow = l16[i]
                r = q * 16 + i
                for j in range(NV):
                    plsc.addupdate(
                        acc.at[lrow, pl.ds(j * 16, 16)],
                        rowbuf[ri, r, pl.ds(j * 16, 16)] * wv)
            return carry
        lax.fori_loop(0, WC // 16, _g16, None)

    fire_scan(0)
    fire_scan(1)

    def _segment(sg, offs):
        def _chunk(kk, offs):
            k = sg * SEG + kk
            wait_scan(k)
            sl = lax.rem(k, 2)

            def _step(q, offs):
                offs = list(offs)
                for ch in range(KCH):
                    eb = sl * SCN + (q * KCH + ch) * 16
                    d16 = dbuf[pl.ds(eb, 16)]
                    s16 = sbuf[pl.ds(eb, 16)]
                    w16 = wbuf[pl.ds(eb, 16)]
                    keep = jnp.logical_and(d16 >= lo, d16 < hi)
                    lb = ch * LR + offs[ch]
                    plsc.store_compressed(lsrc.at[pl.ds(lb, 16)], s16,
                                          mask=keep)
                    plsc.store_compressed(lw.at[pl.ds(lb, 16)], w16,
                                          mask=keep)
                    plsc.store_compressed(ldl.at[pl.ds(lb, 16)], d16 - lo,
                                          mask=keep)
                    cnt = plsc.all_reduce_population_count(keep)
                    cnt = cnt[0] if getattr(cnt, "ndim", 0) else cnt
                    offs[ch] = lax.min(offs[ch] + cnt, jnp.int32(LR - 16))
                return tuple(offs)
            offs = lax.fori_loop(0, SCN // (16 * KCH), _step, offs)

            @pl.when(k + 2 < NCHK)
            def _():
                fire_scan(k + 2)
            return offs
        offs = lax.fori_loop(0, SEG, _chunk, offs)

        nf = [lax.div(offs[ch], jnp.int32(WC)) for ch in range(KCH)]
        c1 = nf[0]
        c2 = c1 + nf[1]
        c3 = c2 + nf[2]
        total = c3 + nf[3]

        def woff(t):
            b = jnp.where(
                t < c1, t * WC,
                jnp.where(t < c2, LR + (t - c1) * WC,
                          jnp.where(t < c3, 2 * LR + (t - c2) * WC,
                                    3 * LR + (t - c3) * WC)))
            return pl.multiple_of(b, 8)

        for t0 in range(RB):
            @pl.when(total >= t0 + 1)
            def _(t0=t0):
                fire_gather(t0, woff(jnp.int32(t0)))

        @pl.loop(0, total)
        def _win(p):
            wait_gather(p)
            process_window(p, woff(p))

            @pl.when(p + RB < total)
            def _():
                fire_gather(p + RB, woff(p + RB))

        rems = []
        for ch in range(KCH):
            rem = offs[ch] - nf[ch] * WC
            rems.append(rem)

            @pl.when(nf[ch] > 0)
            def _carry_rem(ch=ch, tail=nf[ch] * WC):
                tailb = pl.multiple_of(ch * LR + tail, 8)
                for t in range(WC // 16):
                    v_s = lsrc[pl.ds(tailb + t * 16, 16)]
                    v_w = lw[pl.ds(tailb + t * 16, 16)]
                    v_l = ldl[pl.ds(tailb + t * 16, 16)]
                    lsrc[pl.ds(ch * LR + t * 16, 16)] = v_s
                    lw[pl.ds(ch * LR + t * 16, 16)] = v_w
                    ldl[pl.ds(ch * LR + t * 16, 16)] = v_l
        return tuple(rems)
    offs = lax.fori_loop(0, NSEG, _segment,
                         (jnp.int32(0),) * KCH)

    for ch in range(KCH):
        @pl.when(offs[ch] > 0)
        def _drain(ch=ch):
            off = offs[ch]
            zi = jnp.zeros((16,), jnp.int32)
            zf = jnp.zeros((16,), jnp.float32)
            for t in range(6):
                lsrc[pl.ds(ch * LR + off + t * 16, 16)] = zi
                ldl[pl.ds(ch * LR + off + t * 16, 16)] = zi
                lw[pl.ds(ch * LR + off + t * 16, 16)] = zf
            fire_gather(0, jnp.int32(ch * LR))
            wait_gather(0)
            process_window(0, jnp.int32(ch * LR))

    for i in range(RPW // 104):
        pltpu.sync_copy(acc.at[pl.ds(i * 104, 104)],
                        out_hbm.at[pl.ds(lo + i * 104, 104)])

    @pl.when(wid == NW - 1)
    def _tail():
        pltpu.sync_copy(acc.at[pl.ds(RPW, RLAST - RPW)],
                        out_hbm.at[pl.ds(lo + RPW, RLAST - RPW)])


def _sc_aggregate(h, src, dst, w):
    mesh = plsc.VectorSubcoreMesh(core_axis_name="c", subcore_axis_name="s")
    run = pl.kernel(
        _agg_body,
        out_type=jax.ShapeDtypeStruct((N, D), jnp.float32),
        mesh=mesh,
        compiler_params=pltpu.CompilerParams(needs_layout_passes=False),
        scratch_types=[
            pltpu.VMEM((2 * SCN,), jnp.int32),
            pltpu.VMEM((2 * SCN,), jnp.int32),
            pltpu.VMEM((2 * SCN,), jnp.float32),
            pltpu.VMEM((LCAP,), jnp.int32),
            pltpu.VMEM((LCAP,), jnp.int32),
            pltpu.VMEM((LCAP,), jnp.float32),
            pltpu.VMEM((RB, WC, D), jnp.float32),
            pltpu.VMEM((RLAST, D), jnp.float32),
            pltpu.SemaphoreType.DMA((2,)),
            pltpu.SemaphoreType.DMA((RB,)),
        ],
    )
    return run(h, src, dst, w)


@jax.jit
def kernel(x, edge_index, edge_weight, W1, b1, W2, b2):
    src = edge_index[0]
    dst = edge_index[1]
    h1 = _matmul(x, W1)
    a1 = _sc_aggregate(h1, src, dst, edge_weight)
    h2 = _fused_relu_mm(a1, b1, W2)
    a2 = _sc_aggregate(h2, src, dst, edge_weight)
    return _add_bias(a2, b2)


# speedup vs baseline: 1.9967x; 1.9967x over previous
import jax
import jax.numpy as jnp
from jax import lax
from jax.experimental import pallas as pl
from jax.experimental.pallas import tpu as pltpu
from jax.experimental.pallas import tpu_sc as plsc

N = 10000
E = 320000
D = 128

NC = 2
NS = 16
NW = NC * NS
EP = E // NW
C = 80
NCH = EP // C
RPT = 624
ZR = 104
TAIL = N - NS * RPT
RB = 3
IB = 4


def _mm_body(x_ref, w_ref, o_ref):
    o_ref[...] = jnp.dot(x_ref[...], w_ref[...],
                         preferred_element_type=jnp.float32)


def _matmul(x, W, bm=2000):
    n, k = x.shape
    m = W.shape[1]
    return pl.pallas_call(
        _mm_body,
        grid=(n // bm,),
        in_specs=[pl.BlockSpec((bm, k), lambda i: (i, 0)),
                  pl.BlockSpec((k, m), lambda i: (0, 0))],
        out_specs=pl.BlockSpec((bm, m), lambda i: (i, 0)),
        out_shape=jax.ShapeDtypeStruct((n, m), jnp.float32),
    )(x, W)


def _fused_body(p0_ref, p1_ref, b_ref, w_ref, o_ref):
    h = jnp.maximum(p0_ref[...] + p1_ref[...] + b_ref[...], 0.0)
    o_ref[...] = jnp.dot(h, w_ref[...], preferred_element_type=jnp.float32)


def _fused_relu_mm(p0, p1, b, W, bm=2000):
    n, k = p0.shape
    m = W.shape[1]
    return pl.pallas_call(
        _fused_body,
        grid=(n // bm,),
        in_specs=[pl.BlockSpec((bm, k), lambda i: (i, 0)),
                  pl.BlockSpec((bm, k), lambda i: (i, 0)),
                  pl.BlockSpec((1, k), lambda i: (0, 0)),
                  pl.BlockSpec((k, m), lambda i: (0, 0))],
        out_specs=pl.BlockSpec((bm, m), lambda i: (i, 0)),
        out_shape=jax.ShapeDtypeStruct((n, m), jnp.float32),
    )(p0, p1, b.reshape(1, k), W)


def _bias_body(p0_ref, p1_ref, b_ref, o_ref):
    o_ref[...] = p0_ref[...] + p1_ref[...] + b_ref[...]


def _add_partials_bias(p0, p1, b, bm=2000):
    n, k = p0.shape
    return pl.pallas_call(
        _bias_body,
        grid=(n // bm,),
        in_specs=[pl.BlockSpec((bm, k), lambda i: (i, 0)),
                  pl.BlockSpec((bm, k), lambda i: (i, 0)),
                  pl.BlockSpec((1, k), lambda i: (0, 0))],
        out_specs=pl.BlockSpec((bm, k), lambda i: (i, 0)),
        out_shape=jax.ShapeDtypeStruct((n, k), jnp.float32),
    )(p0, p1, b.reshape(1, k))


def _agg_body(h_hbm, src_hbm, dst_hbm, w_hbm, out_hbm,
              srcbuf, dstbuf, wbuf, rowbuf, zbuf, agg, isem, gsem, ssem):
    c = lax.axis_index("c")
    s = lax.axis_index("s")
    wid = c * NS + s

    def _zrow(r, carry):
        zero = jnp.zeros((16,), jnp.float32)
        for j in range(D // 16):
            zbuf[r, pl.ds(j * 16, 16)] = zero
        return carry
    lax.fori_loop(0, ZR, _zrow, None)
    row0 = s * RPT
    for i in range(RPT // ZR):
        pltpu.sync_copy(zbuf, agg.at[pl.ds(row0 + i * ZR, ZR)])

    @pl.when(s == NS - 1)
    def _zero_tail():
        pltpu.sync_copy(zbuf.at[pl.ds(0, TAIL)], agg.at[pl.ds(NS * RPT, TAIL)])
    plsc.subcore_barrier()

    def fire_idx(g):
        sl = lax.rem(g, IB)
        base = wid * EP + g * C
        pltpu.async_copy(src_hbm.at[pl.ds(base, C)], srcbuf.at[sl], isem.at[sl])
        pltpu.async_copy(dst_hbm.at[pl.ds(base, C)], dstbuf.at[sl], isem.at[sl])
        pltpu.async_copy(w_hbm.at[pl.ds(base, C)], wbuf.at[sl], isem.at[sl])

    def wait_idx(g):
        sl = lax.rem(g, IB)
        pltpu.make_async_copy(src_hbm.at[pl.ds(0, C)], srcbuf.at[sl],
                              isem.at[sl]).wait()
        pltpu.make_async_copy(dst_hbm.at[pl.ds(0, C)], dstbuf.at[sl],
                              isem.at[sl]).wait()
        pltpu.make_async_copy(w_hbm.at[pl.ds(0, C)], wbuf.at[sl],
                              isem.at[sl]).wait()

    def fire_gather(g):
        ri = lax.rem(g, RB)
        ii = lax.rem(g, IB)
        pltpu.async_copy(h_hbm.at[srcbuf.at[ii]], rowbuf.at[ri], gsem.at[ri])

    def wait_gather(g):
        ri = lax.rem(g, RB)
        ii = lax.rem(g, IB)
        pltpu.make_async_copy(h_hbm.at[srcbuf.at[ii]], rowbuf.at[ri],
                              gsem.at[ri]).wait()

    def fire_scatter(g):
        ri = lax.rem(g, RB)
        ii = lax.rem(g, IB)
        pltpu.async_copy(rowbuf.at[ri], agg.at[dstbuf.at[ii]], ssem.at[ri],
                         add=True)

    def wait_scatter(g):
        ri = lax.rem(g, RB)
        ii = lax.rem(g, IB)
        pltpu.make_async_copy(rowbuf.at[ri], agg.at[dstbuf.at[ii]],
                              ssem.at[ri]).wait()

    fire_idx(0)
    fire_idx(1)
    wait_idx(0)
    fire_gather(0)

    def _chunk(g, carry):
        @pl.when(g >= 2)
        def _():
            wait_scatter(g - 2)

        @pl.when(g + 1 < NCH)
        def _():
            wait_idx(g + 1)
            fire_gather(g + 1)

        @pl.when(g + 2 < NCH)
        def _():
            fire_idx(g + 2)

        wait_gather(g)
        ri = lax.rem(g, RB)
        ii = lax.rem(g, IB)

        def _q(q, carry2):
            w16 = wbuf[ii, pl.ds(q * 16, 16)]
            for i in range(16):
                wv = jnp.full((16,), w16[i], jnp.float32)
                r = q * 16 + i
                for j in range(D // 16):
                    rowbuf[ri, r, pl.ds(j * 16, 16)] = (
                        rowbuf[ri, r, pl.ds(j * 16, 16)] * wv)
            return carry2
        lax.fori_loop(0, C // 16, _q, None)

        fire_scatter(g)
        return carry
    lax.fori_loop(0, NCH, _chunk, None)
    wait_scatter(NCH - 2)
    wait_scatter(NCH - 1)
    plsc.subcore_barrier()

    for i in range(RPT // ZR):
        r0 = row0 + i * ZR
        pltpu.sync_copy(agg.at[pl.ds(r0, ZR)], out_hbm.at[c, pl.ds(r0, ZR)])

    @pl.when(s == NS - 1)
    def _copy_tail():
        pltpu.sync_copy(agg.at[pl.ds(NS * RPT, TAIL)],
                        out_hbm.at[c, pl.ds(NS * RPT, TAIL)])


def _sc_aggregate(h, src, dst, edge_weight):
    mesh = plsc.VectorSubcoreMesh(core_axis_name="c", subcore_axis_name="s")
    run = pl.kernel(
        _agg_body,
        out_type=jax.ShapeDtypeStruct((NC, N, D), jnp.float32),
        mesh=mesh,
        scratch_types=[
            pltpu.VMEM((IB, C), jnp.int32),
            pltpu.VMEM((IB, C), jnp.int32),
            pltpu.VMEM((IB, C), jnp.float32),
            pltpu.VMEM((RB, C, D), jnp.float32),
            pltpu.VMEM((ZR, D), jnp.float32),
            pltpu.VMEM_SHARED((N, D), jnp.float32),
            pltpu.SemaphoreType.DMA((IB,)),
            pltpu.SemaphoreType.DMA((RB,)),
            pltpu.SemaphoreType.DMA((RB,)),
        ],
    )
    return run(h, src, dst, edge_weight)


@jax.jit
def kernel(x, edge_index, edge_weight, W1, b1, W2, b2):
    src = edge_index[0]
    dst = edge_index[1]
    h1 = _matmul(x, W1)
    p1 = _sc_aggregate(h1, src, dst, edge_weight)
    h2 = _fused_relu_mm(p1[0], p1[1], b1, W2)
    p2 = _sc_aggregate(h2, src, dst, edge_weight)
    return _add_partials_bias(p2[0], p2[1], b2)
